# baseline (speedup 1.0000x reference)
import functools

import jax
import jax.numpy as jnp
from jax import lax
from jax.experimental import pallas as pl
from jax.experimental.pallas import tpu as pltpu
from jax.experimental.pallas import tpu_sc as plsc

_INPUT_DIM = 768
_NUM_EXPERTS = 8
_TOPK = 2
_LOAD_BALANCING_COEF = 0.1

_N_TOKENS = 32768
_NCHUNK = 2
_CHUNK = _N_TOKENS // _NCHUNK
_BLK = 4096
_NC, _NS, _LANES = 2, 16, 16
_NTILES = _NC * _NS
_TPT = _CHUNK // _NTILES
_GROUPS = _TPT // _LANES


def _logits_body(x_ref, w_ref, out_ref):
    out_ref[...] = lax.dot_general(
        w_ref[...], x_ref[...], (((1,), (1,)), ((), ())),
        preferred_element_type=jnp.float32)


def _logits_chunk(x, w):
    grid = (_CHUNK // _BLK,)
    return pl.pallas_call(
        _logits_body,
        grid=grid,
        in_specs=[
            pl.BlockSpec((_BLK, _INPUT_DIM), lambda i: (i, 0)),
            pl.BlockSpec((_NUM_EXPERTS, _INPUT_DIM), lambda i: (0, 0)),
        ],
        out_specs=pl.BlockSpec((_NUM_EXPERTS, _BLK), lambda i: (0, i)),
        out_shape=jax.ShapeDtypeStruct((_NUM_EXPERTS, _CHUNK), jnp.float32),
        compiler_params=pltpu.CompilerParams(
            dimension_semantics=("arbitrary",)),
    )(x, w)


def _route_body(lg_hbm, p1_hbm, p2_hbm, i1_hbm, i2_hbm, aggp_hbm, cntp_hbm,
                lg_v, p1_v, p2_v, i1_v, i2_v, agg_v, cnt_v):
    wid = lax.axis_index("s") * _NC + lax.axis_index("c")
    base = wid * _TPT
    pltpu.sync_copy(lg_hbm.at[:, pl.ds(base, _TPT)], lg_v)

    zero_f = jnp.zeros((_LANES,), jnp.float32)
    one_f = jnp.ones((_LANES,), jnp.float32)

    def group(g, carry):
        aggs, cnts = carry
        off = g * _LANES
        ls = [lg_v[e, pl.ds(off, _LANES)] for e in range(_NUM_EXPERTS)]
        m = ls[0]
        for e in range(1, _NUM_EXPERTS):
            m = jnp.maximum(m, ls[e])
        es = [jnp.exp(l - m) for l in ls]
        s = es[0]
        for e in range(1, _NUM_EXPERTS):
            s = s + es[e]
        r = one_f / s
        ps = [e_ * r for e_ in es]

        m1 = ps[0]
        i1 = jnp.zeros((_LANES,), jnp.int32)
        m2 = jnp.full((_LANES,), -1.0, jnp.float32)
        i2 = jnp.zeros((_LANES,), jnp.int32)
        for e in range(1, _NUM_EXPERTS):
            p = ps[e]
            ei = jnp.full((_LANES,), e, jnp.int32)
            gt1 = p > m1
            gt2 = p > m2
            i2 = jnp.where(gt1, i1, jnp.where(gt2, ei, i2))
            m2 = jnp.where(gt1, m1, jnp.where(gt2, p, m2))
            i1 = jnp.where(gt1, ei, i1)
            m1 = jnp.where(gt1, p, m1)

        p1_v[pl.ds(off, _LANES)] = m1
        p2_v[pl.ds(off, _LANES)] = m2
        i1_v[pl.ds(off, _LANES)] = i1
        i2_v[pl.ds(off, _LANES)] = i2

        new_aggs = tuple(a + p for a, p in zip(aggs, ps))
        new_cnts = tuple(
            c + jnp.where(i1 == e, one_f, zero_f)
            + jnp.where(i2 == e, one_f, zero_f)
            for e, c in enumerate(cnts))
        return new_aggs, new_cnts

    init = (tuple(zero_f for _ in range(_NUM_EXPERTS)),
            tuple(zero_f for _ in range(_NUM_EXPERTS)))
    aggs, cnts = lax.fori_loop(0, _GROUPS, group, init)

    for e in range(_NUM_EXPERTS):
        agg_v[pl.ds(e * _LANES, _LANES)] = aggs[e]
        cnt_v[pl.ds(e * _LANES, _LANES)] = cnts[e]

    pltpu.sync_copy(p1_v, p1_hbm.at[pl.ds(base, _TPT)])
    pltpu.sync_copy(p2_v, p2_hbm.at[pl.ds(base, _TPT)])
    pltpu.sync_copy(i1_v, i1_hbm.at[pl.ds(base, _TPT)])
    pltpu.sync_copy(i2_v, i2_hbm.at[pl.ds(base, _TPT)])
    pltpu.sync_copy(agg_v, aggp_hbm.at[wid])
    pltpu.sync_copy(cnt_v, cntp_hbm.at[wid])


_route_sc = pl.kernel(
    _route_body,
    out_type=(
        jax.ShapeDtypeStruct((_CHUNK,), jnp.float32),
        jax.ShapeDtypeStruct((_CHUNK,), jnp.float32),
        jax.ShapeDtypeStruct((_CHUNK,), jnp.int32),
        jax.ShapeDtypeStruct((_CHUNK,), jnp.int32),
        jax.ShapeDtypeStruct((_NTILES, _NUM_EXPERTS * _LANES), jnp.float32),
        jax.ShapeDtypeStruct((_NTILES, _NUM_EXPERTS * _LANES), jnp.float32),
    ),
    mesh=plsc.VectorSubcoreMesh(core_axis_name="c", subcore_axis_name="s"),
    scratch_types=[
        pltpu.VMEM((_NUM_EXPERTS, _TPT), jnp.float32),
        pltpu.VMEM((_TPT,), jnp.float32),
        pltpu.VMEM((_TPT,), jnp.float32),
        pltpu.VMEM((_TPT,), jnp.int32),
        pltpu.VMEM((_TPT,), jnp.int32),
        pltpu.VMEM((_NUM_EXPERTS * _LANES,), jnp.float32),
        pltpu.VMEM((_NUM_EXPERTS * _LANES,), jnp.float32),
    ],
)


def _loss_body(agg0_ref, cnt0_ref, agg1_ref, cnt1_ref, loss_ref):
    agg = (jnp.sum(agg0_ref[...], axis=0, keepdims=True)
           + jnp.sum(agg1_ref[...], axis=0, keepdims=True))
    cnt = (jnp.sum(cnt0_ref[...], axis=0, keepdims=True)
           + jnp.sum(cnt1_ref[...], axis=0, keepdims=True))
    n = _NUM_EXPERTS * _LANES
    row = lax.broadcasted_iota(jnp.int32, (n, n), 0) // _LANES
    col = lax.broadcasted_iota(jnp.int32, (n, n), 1) // _LANES
    seg = (row == col).astype(jnp.float32)
    segcnt = lax.dot_general(cnt, seg, (((1,), (0,)), ((), ())),
                             preferred_element_type=jnp.float32)
    scale = (_NUM_EXPERTS * _LOAD_BALANCING_COEF
             / (_N_TOKENS * _N_TOKENS * _TOPK))
    loss_ref[...] = (jnp.sum(agg * segcnt) * scale).reshape(1, 1)


def _loss(agg0, cnt0, agg1, cnt1):
    n = _NUM_EXPERTS * _LANES
    spec = pl.BlockSpec((_NTILES, n), lambda: (0, 0))
    return pl.pallas_call(
        _loss_body,
        in_specs=[spec, spec, spec, spec],
        out_specs=pl.BlockSpec((1, 1), lambda: (0, 0)),
        out_shape=jax.ShapeDtypeStruct((1, 1), jnp.float32),
    )(agg0, cnt0, agg1, cnt1)


@jax.jit
def _router(x, w):
    x0 = x[:_CHUNK]
    x1 = x[_CHUNK:]
    lg0 = _logits_chunk(x0, w)
    lg1 = _logits_chunk(x1, w)
    p1a, p2a, i1a, i2a, agg0, cnt0 = _route_sc(lg0)
    p1b, p2b, i1b, i2b, agg1, cnt1 = _route_sc(lg1)
    loss = _loss(agg0, cnt0, agg1, cnt1)
    p1 = jnp.concatenate([p1a, p1b])
    p2 = jnp.concatenate([p2a, p2b])
    i1 = jnp.concatenate([i1a, i1b])
    i2 = jnp.concatenate([i2a, i2b])
    top_probs = jnp.stack([p1, p2], axis=1)
    top_indices = jnp.stack([i1, i2], axis=1)
    return top_probs, top_indices, loss[0, 0]


def kernel(input, W):
    x = input.reshape(-1, _INPUT_DIM)
    return _router(x, W)


# speedup vs baseline: 2.2002x; 2.2002x over previous
import functools

import jax
import jax.numpy as jnp
from jax import lax
from jax.experimental import pallas as pl
from jax.experimental.pallas import tpu as pltpu
from jax.experimental.pallas import tpu_sc as plsc

_INPUT_DIM = 768
_NUM_EXPERTS = 8
_TOPK = 2
_LOAD_BALANCING_COEF = 0.1

_N_TOKENS = 32768
_BLK = 4096
_NC, _NS, _LANES = 2, 16, 16
_NTILES = _NC * _NS
_TPT = _N_TOKENS // _NTILES
_GROUPS = _TPT // _LANES


def _logits_body(x_ref, w_ref, out_ref):
    out_ref[...] = lax.dot_general(
        w_ref[...], x_ref[...], (((1,), (1,)), ((), ())),
        preferred_element_type=jnp.float32)


def _logits_t(x, w):
    grid = (_N_TOKENS // _BLK,)
    return pl.pallas_call(
        _logits_body,
        grid=grid,
        in_specs=[
            pl.BlockSpec((_BLK, _INPUT_DIM), lambda i: (i, 0)),
            pl.BlockSpec((_NUM_EXPERTS, _INPUT_DIM), lambda i: (0, 0)),
        ],
        out_specs=pl.BlockSpec((_NUM_EXPERTS, _BLK), lambda i: (0, i)),
        out_shape=jax.ShapeDtypeStruct((_NUM_EXPERTS, _N_TOKENS), jnp.float32),
        compiler_params=pltpu.CompilerParams(
            dimension_semantics=("arbitrary",)),
    )(x, w)


def _route_body(lg_hbm, pq_hbm, idx_hbm, parts_hbm,
                lg_v, p1_v, p2_v, i1_v, i2_v, agg_v, cnt_v):
    wid = lax.axis_index("s") * _NC + lax.axis_index("c")
    base = wid * _TPT
    pltpu.sync_copy(lg_hbm.at[:, pl.ds(base, _TPT)], lg_v)

    zero_f = jnp.zeros((_LANES,), jnp.float32)
    one_f = jnp.ones((_LANES,), jnp.float32)

    def group(g, carry):
        aggs, cnts = carry
        off = g * _LANES
        ls = [lg_v[e, pl.ds(off, _LANES)] for e in range(_NUM_EXPERTS)]
        m = ls[0]
        for e in range(1, _NUM_EXPERTS):
            m = jnp.maximum(m, ls[e])
        es = [jnp.exp(l - m) for l in ls]
        s = es[0]
        for e in range(1, _NUM_EXPERTS):
            s = s + es[e]
        r = one_f / s
        ps = [e_ * r for e_ in es]

        m1 = ps[0]
        i1 = jnp.zeros((_LANES,), jnp.int32)
        m2 = jnp.full((_LANES,), -1.0, jnp.float32)
        i2 = jnp.zeros((_LANES,), jnp.int32)
        for e in range(1, _NUM_EXPERTS):
            p = ps[e]
            ei = jnp.full((_LANES,), e, jnp.int32)
            gt1 = p > m1
            gt2 = p > m2
            i2 = jnp.where(gt1, i1, jnp.where(gt2, ei, i2))
            m2 = jnp.where(gt1, m1, jnp.where(gt2, p, m2))
            i1 = jnp.where(gt1, ei, i1)
            m1 = jnp.where(gt1, p, m1)

        p1_v[pl.ds(off, _LANES)] = m1
        p2_v[pl.ds(off, _LANES)] = m2
        i1_v[pl.ds(off, _LANES)] = i1
        i2_v[pl.ds(off, _LANES)] = i2

        new_aggs = tuple(a + p for a, p in zip(aggs, ps))
        new_cnts = tuple(
            c + jnp.where(i1 == e, one_f, zero_f)
            + jnp.where(i2 == e, one_f, zero_f)
            for e, c in enumerate(cnts))
        return new_aggs, new_cnts

    init = (tuple(zero_f for _ in range(_NUM_EXPERTS)),
            tuple(zero_f for _ in range(_NUM_EXPERTS)))
    aggs, cnts = lax.fori_loop(0, _GROUPS, group, init)

    for e in range(_NUM_EXPERTS):
        agg_v[pl.ds(e * _LANES, _LANES)] = aggs[e]
        cnt_v[pl.ds(e * _LANES, _LANES)] = cnts[e]

    pltpu.sync_copy(p1_v, pq_hbm.at[0, pl.ds(base, _TPT)])
    pltpu.sync_copy(p2_v, pq_hbm.at[1, pl.ds(base, _TPT)])
    pltpu.sync_copy(i1_v, idx_hbm.at[0, pl.ds(base, _TPT)])
    pltpu.sync_copy(i2_v, idx_hbm.at[1, pl.ds(base, _TPT)])
    pltpu.sync_copy(agg_v, parts_hbm.at[0, wid])
    pltpu.sync_copy(cnt_v, parts_hbm.at[1, wid])


_route_sc = pl.kernel(
    _route_body,
    out_type=(
        jax.ShapeDtypeStruct((2, _N_TOKENS), jnp.float32),
        jax.ShapeDtypeStruct((2, _N_TOKENS), jnp.int32),
        jax.ShapeDtypeStruct((2, _NTILES, _NUM_EXPERTS * _LANES),
                             jnp.float32),
    ),
    mesh=plsc.VectorSubcoreMesh(core_axis_name="c", subcore_axis_name="s"),
    scratch_types=[
        pltpu.VMEM((_NUM_EXPERTS, _TPT), jnp.float32),
        pltpu.VMEM((_TPT,), jnp.float32),
        pltpu.VMEM((_TPT,), jnp.float32),
        pltpu.VMEM((_TPT,), jnp.int32),
        pltpu.VMEM((_TPT,), jnp.int32),
        pltpu.VMEM((_NUM_EXPERTS * _LANES,), jnp.float32),
        pltpu.VMEM((_NUM_EXPERTS * _LANES,), jnp.float32),
    ],
)


def _loss_body(parts_ref, loss_ref):
    agg = jnp.sum(parts_ref[0], axis=0, keepdims=True)
    cnt = jnp.sum(parts_ref[1], axis=0, keepdims=True)
    n = _NUM_EXPERTS * _LANES
    row = lax.broadcasted_iota(jnp.int32, (n, n), 0) // _LANES
    col = lax.broadcasted_iota(jnp.int32, (n, n), 1) // _LANES
    seg = (row == col).astype(jnp.float32)
    segcnt = lax.dot_general(cnt, seg, (((1,), (0,)), ((), ())),
                             preferred_element_type=jnp.float32)
    scale = (_NUM_EXPERTS * _LOAD_BALANCING_COEF
             / (_N_TOKENS * _N_TOKENS * _TOPK))
    loss_ref[...] = (jnp.sum(agg * segcnt) * scale).reshape(1, 1)


def _loss(parts):
    n = _NUM_EXPERTS * _LANES
    spec = pl.BlockSpec((2, _NTILES, n), lambda: (0, 0, 0))
    return pl.pallas_call(
        _loss_body,
        in_specs=[spec],
        out_specs=pl.BlockSpec((1, 1), lambda: (0, 0)),
        out_shape=jax.ShapeDtypeStruct((1, 1), jnp.float32),
    )(parts)


@jax.jit
def _router(x, w):
    lg = _logits_t(x, w)
    pq, idx, parts = _route_sc(lg)
    loss = _loss(parts)
    top_probs = pq.T
    top_indices = idx.T
    return top_probs, top_indices, loss[0, 0]


def kernel(input, W):
    x = input.reshape(-1, _INPUT_DIM)
    return _router(x, W)


# speedup vs baseline: 2.2034x; 1.0014x over previous
import jax
import jax.numpy as jnp
from jax import lax
from jax.experimental import pallas as pl
from jax.experimental.pallas import tpu as pltpu
from jax.experimental.pallas import tpu_sc as plsc

_INPUT_DIM = 768
_NUM_EXPERTS = 8
_TOPK = 2
_LOAD_BALANCING_COEF = 0.1

_N_TOKENS = 32768
_BLK = 4096
_NC, _NS, _LANES = 2, 16, 16
_NTILES = _NC * _NS
_TPT = _N_TOKENS // _NTILES
_GROUPS = _TPT // _LANES


def _logits_body(x_ref, w_ref, out_ref):
    out_ref[...] = lax.dot_general(
        w_ref[...], x_ref[...], (((1,), (1,)), ((), ())),
        preferred_element_type=jnp.float32)


def _logits_t(x, w):
    grid = (_N_TOKENS // _BLK,)
    return pl.pallas_call(
        _logits_body,
        grid=grid,
        in_specs=[
            pl.BlockSpec((_BLK, _INPUT_DIM), lambda i: (i, 0)),
            pl.BlockSpec((_NUM_EXPERTS, _INPUT_DIM), lambda i: (0, 0)),
        ],
        out_specs=pl.BlockSpec((_NUM_EXPERTS, _BLK), lambda i: (0, i)),
        out_shape=jax.ShapeDtypeStruct((_NUM_EXPERTS, _N_TOKENS), jnp.float32),
        compiler_params=pltpu.CompilerParams(
            dimension_semantics=("arbitrary",)),
    )(x, w)


def _route_body(lg_hbm, pq_hbm, idx_hbm, parts_hbm,
                lg_v, p1_v, p2_v, i1_v, i2_v, agg_v, cnt_v):
    wid = lax.axis_index("s") * _NC + lax.axis_index("c")
    base = wid * _TPT
    pltpu.sync_copy(lg_hbm.at[:, pl.ds(base, _TPT)], lg_v)

    zero_f = jnp.zeros((_LANES,), jnp.float32)
    one_f = jnp.ones((_LANES,), jnp.float32)

    def group(g, carry):
        aggs, cnts = carry
        off = g * _LANES
        ls = [lg_v[e, pl.ds(off, _LANES)] for e in range(_NUM_EXPERTS)]
        m = ls[0]
        for e in range(1, _NUM_EXPERTS):
            m = jnp.maximum(m, ls[e])
        es = [jnp.exp(l - m) for l in ls]
        s = es[0]
        for e in range(1, _NUM_EXPERTS):
            s = s + es[e]
        r = one_f / s
        ps = [e_ * r for e_ in es]

        m1 = ps[0]
        i1 = jnp.zeros((_LANES,), jnp.int32)
        m2 = jnp.full((_LANES,), -1.0, jnp.float32)
        i2 = jnp.zeros((_LANES,), jnp.int32)
        for e in range(1, _NUM_EXPERTS):
            p = ps[e]
            ei = jnp.full((_LANES,), e, jnp.int32)
            gt1 = p > m1
            gt2 = p > m2
            i2 = jnp.where(gt1, i1, jnp.where(gt2, ei, i2))
            m2 = jnp.where(gt1, m1, jnp.where(gt2, p, m2))
            i1 = jnp.where(gt1, ei, i1)
            m1 = jnp.where(gt1, p, m1)

        p1_v[pl.ds(off, _LANES)] = m1
        p2_v[pl.ds(off, _LANES)] = m2
        i1_v[pl.ds(off, _LANES)] = i1
        i2_v[pl.ds(off, _LANES)] = i2

        new_aggs = tuple(a + p for a, p in zip(aggs, ps))
        new_cnts = tuple(
            c + jnp.where(i1 == e, one_f, zero_f)
            + jnp.where(i2 == e, one_f, zero_f)
            for e, c in enumerate(cnts))
        return new_aggs, new_cnts

    init = (tuple(zero_f for _ in range(_NUM_EXPERTS)),
            tuple(zero_f for _ in range(_NUM_EXPERTS)))
    aggs, cnts = lax.fori_loop(0, _GROUPS, group, init)

    for e in range(_NUM_EXPERTS):
        agg_v[pl.ds(e * _LANES, _LANES)] = aggs[e]
        cnt_v[pl.ds(e * _LANES, _LANES)] = cnts[e]

    pltpu.sync_copy(p1_v, pq_hbm.at[0, pl.ds(base, _TPT)])
    pltpu.sync_copy(p2_v, pq_hbm.at[1, pl.ds(base, _TPT)])
    pltpu.sync_copy(i1_v, idx_hbm.at[0, pl.ds(base, _TPT)])
    pltpu.sync_copy(i2_v, idx_hbm.at[1, pl.ds(base, _TPT)])
    pltpu.sync_copy(agg_v, parts_hbm.at[0, wid])
    pltpu.sync_copy(cnt_v, parts_hbm.at[1, wid])


_route_sc = pl.kernel(
    _route_body,
    out_type=(
        jax.ShapeDtypeStruct((2, _N_TOKENS), jnp.float32),
        jax.ShapeDtypeStruct((2, _N_TOKENS), jnp.int32),
        jax.ShapeDtypeStruct((2, _NTILES, _NUM_EXPERTS * _LANES),
                             jnp.float32),
    ),
    mesh=plsc.VectorSubcoreMesh(core_axis_name="c", subcore_axis_name="s"),
    scratch_types=[
        pltpu.VMEM((_NUM_EXPERTS, _TPT), jnp.float32),
        pltpu.VMEM((_TPT,), jnp.float32),
        pltpu.VMEM((_TPT,), jnp.float32),
        pltpu.VMEM((_TPT,), jnp.int32),
        pltpu.VMEM((_TPT,), jnp.int32),
        pltpu.VMEM((_NUM_EXPERTS * _LANES,), jnp.float32),
        pltpu.VMEM((_NUM_EXPERTS * _LANES,), jnp.float32),
    ],
)


def _loss_body(parts_ref, loss_ref):
    agg = jnp.sum(parts_ref[0], axis=0, keepdims=True)
    cnt = jnp.sum(parts_ref[1], axis=0, keepdims=True)
    n = _NUM_EXPERTS * _LANES
    row = lax.broadcasted_iota(jnp.int32, (n, n), 0) // _LANES
    col = lax.broadcasted_iota(jnp.int32, (n, n), 1) // _LANES
    seg = (row == col).astype(jnp.float32)
    segcnt = lax.dot_general(cnt, seg, (((1,), (0,)), ((), ())),
                             preferred_element_type=jnp.float32)
    scale = (_NUM_EXPERTS * _LOAD_BALANCING_COEF
             / (_N_TOKENS * _N_TOKENS * _TOPK))
    loss_ref[...] = (jnp.sum(agg * segcnt) * scale).reshape(1, 1)


def _loss(parts):
    n = _NUM_EXPERTS * _LANES
    spec = pl.BlockSpec((2, _NTILES, n), lambda: (0, 0, 0))
    return pl.pallas_call(
        _loss_body,
        in_specs=[spec],
        out_specs=pl.BlockSpec((1, 1), lambda: (0, 0)),
        out_shape=jax.ShapeDtypeStruct((1, 1), jnp.float32),
    )(parts)


@jax.jit
def _router(x, w):
    lg = _logits_t(x, w)
    pq, idx, parts = _route_sc(lg)
    loss = _loss(parts)
    top_probs = pq.T
    top_indices = idx.T
    return top_probs, top_indices, loss[0, 0]


def kernel(input, W):
    x = input.reshape(-1, _INPUT_DIM)
    return _router(x, W)
